# baseline (speedup 1.0000x reference)
import functools
import math

import jax
import jax.numpy as jnp
import numpy as np
from jax import lax
from jax.experimental import pallas as pl
from jax.experimental.pallas import tpu as pltpu
from jax.experimental.pallas import tpu_sc as plsc

_VOCAB = 50268
_DIM = 384
_MAXLEN = 1024
_PAD = 50257
_SCALE = math.sqrt(_DIM)
_BATCH = 64

_NW = 32
_PPW = _MAXLEN // _NW
_LANES = 16


def _build_pe() -> np.ndarray:
    position = np.arange(_MAXLEN, dtype=np.float64)[:, None]
    div_term = np.exp(
        np.arange(0, _DIM, 2, dtype=np.float64) * (-math.log(10000.0) / _DIM)
    )
    pe = np.zeros((_MAXLEN, _DIM), dtype=np.float32)
    pe[:, 0::2] = np.sin(position * div_term)
    pe[:, 1::2] = np.cos(position * div_term)
    return pe


_PE = _build_pe()

_mesh = plsc.VectorSubcoreMesh(core_axis_name="c", subcore_axis_name="s")


@functools.partial(
    pl.kernel,
    out_type=jax.ShapeDtypeStruct((_BATCH * _MAXLEN, _DIM), jnp.float32),
    mesh=_mesh,
    scratch_types=[
        pltpu.VMEM((_PPW, _DIM), jnp.float32),
        pltpu.VMEM((_PPW,), jnp.int32),
        pltpu.VMEM((_PPW + _LANES,), jnp.float32),
        pltpu.VMEM((_PPW, _DIM), jnp.float32),
        pltpu.SemaphoreType.DMA,
    ],
)
def _emb_kernel(ids_hbm, table_hbm, pe_hbm, out_hbm, pe_v, idx_v, msk_v, rows_v, gsem):
    wid = lax.axis_index("s") * 2 + lax.axis_index("c")
    pbase = wid * _PPW
    pltpu.sync_copy(pe_hbm.at[pl.ds(pbase, _PPW)], pe_v)
    msk_v[pl.ds(_PPW, _LANES)] = jnp.zeros((_LANES,), jnp.float32)

    def chunk(b, carry):
        base = b * _MAXLEN + pbase
        pltpu.sync_copy(ids_hbm.at[pl.ds(base, _PPW)], idx_v)
        for h in range(_PPW // _LANES):
            ids16 = idx_v[pl.ds(h * _LANES, _LANES)]
            msk_v[pl.ds(h * _LANES, _LANES)] = jnp.where(
                ids16 != _PAD, jnp.float32(1.0), jnp.float32(0.0)
            )
        pltpu.async_copy(table_hbm.at[idx_v], rows_v, gsem).wait()

        def tok(t, c):
            m = jnp.full((_LANES,), msk_v[pl.ds(t, _LANES)][0], jnp.float32)
            for j in range(_DIM // _LANES):
                sl = pl.ds(j * _LANES, _LANES)
                v = rows_v[t, sl] * jnp.float32(_SCALE) + pe_v[t, sl]
                rows_v[t, sl] = v * m
            return c

        lax.fori_loop(0, _PPW, tok, 0)
        pltpu.sync_copy(rows_v, out_hbm.at[pl.ds(base, _PPW)])
        return carry

    lax.fori_loop(0, _BATCH, chunk, 0)


def kernel(input_ids, table):
    bsz, seq = input_ids.shape
    ids_flat = input_ids.reshape(-1).astype(jnp.int32)
    pe = jnp.asarray(_PE)
    out = _emb_kernel(ids_flat, table, pe)
    return out.reshape(bsz, seq, _DIM)


# speedup vs baseline: 2.1809x; 2.1809x over previous
import functools
import math

import jax
import jax.numpy as jnp
import numpy as np
from jax import lax
from jax.experimental import pallas as pl
from jax.experimental.pallas import tpu as pltpu
from jax.experimental.pallas import tpu_sc as plsc

_VOCAB = 50268
_DIM = 384
_MAXLEN = 1024
_PAD = 50257
_SCALE = math.sqrt(_DIM)
_BATCH = 64

_NW = 32
_PPW = _MAXLEN // _NW
_LANES = 16
_NSL = _DIM // _LANES


def _build_pe() -> np.ndarray:
    position = np.arange(_MAXLEN, dtype=np.float64)[:, None]
    div_term = np.exp(
        np.arange(0, _DIM, 2, dtype=np.float64) * (-math.log(10000.0) / _DIM)
    )
    pe = np.zeros((_MAXLEN, _DIM), dtype=np.float32)
    pe[:, 0::2] = np.sin(position * div_term)
    pe[:, 1::2] = np.cos(position * div_term)
    return pe


_PE = _build_pe()

_mesh = plsc.VectorSubcoreMesh(core_axis_name="c", subcore_axis_name="s")


@functools.partial(
    pl.kernel,
    out_type=jax.ShapeDtypeStruct((_BATCH, _MAXLEN, _DIM), jnp.float32),
    mesh=_mesh,
    scratch_types=[
        pltpu.VMEM((_PPW, _DIM), jnp.float32),
        pltpu.VMEM((_PPW, _BATCH), jnp.int32),
        pltpu.VMEM((_PPW * _BATCH + _LANES,), jnp.float32),
        pltpu.VMEM((2, _BATCH, _DIM), jnp.float32),
        pltpu.SemaphoreType.DMA,
        pltpu.SemaphoreType.DMA,
    ],
)
def _emb_kernel(ids_hbm, table_hbm, pe_hbm, out_hbm,
                pe_v, idx_v, msk_v, rows_v, gsem, ssem):
    wid = lax.axis_index("s") * 2 + lax.axis_index("c")
    pbase = wid * _PPW
    pltpu.sync_copy(pe_hbm.at[pl.ds(pbase, _PPW)], pe_v)
    pltpu.sync_copy(ids_hbm.at[pl.ds(pbase, _PPW)], idx_v)

    msk_v[pl.ds(_PPW * _BATCH, _LANES)] = jnp.zeros((_LANES,), jnp.float32)

    def mask_row(r, c):
        for c4 in range(_BATCH // _LANES):
            ids16 = idx_v[r, pl.ds(c4 * _LANES, _LANES)]
            msk_v[pl.ds(r * _BATCH + c4 * _LANES, _LANES)] = jnp.where(
                ids16 != _PAD, jnp.float32(1.0), jnp.float32(0.0)
            )
        return c

    lax.fori_loop(0, _PPW, mask_row, 0)

    pltpu.async_copy(table_hbm.at[idx_v.at[0]], rows_v.at[0], gsem)

    def position(t, carry):
        p = lax.rem(t, 2)
        q = 1 - p
        pltpu.make_async_copy(
            table_hbm.at[idx_v.at[t]], rows_v.at[p], gsem
        ).wait()

        @pl.when(t >= 1)
        def _():
            pltpu.make_async_copy(
                rows_v.at[q], out_hbm.at[:, pbase + t - 1, :], ssem
            ).wait()

        @pl.when(t + 1 < _PPW)
        def _():
            pltpu.async_copy(table_hbm.at[idx_v.at[t + 1]], rows_v.at[q], gsem)

        pe_regs = [pe_v[t, pl.ds(j * _LANES, _LANES)] for j in range(_NSL)]

        def token(b, c):
            m = jnp.full(
                (_LANES,), msk_v[pl.ds(t * _BATCH + b, _LANES)][0], jnp.float32
            )
            for j in range(_NSL):
                sl = pl.ds(j * _LANES, _LANES)
                v = rows_v[p, b, sl] * jnp.float32(_SCALE) + pe_regs[j]
                rows_v[p, b, sl] = v * m
            return c

        lax.fori_loop(0, _BATCH, token, 0)
        pltpu.async_copy(rows_v.at[p], out_hbm.at[:, pbase + t, :], ssem)
        return carry

    lax.fori_loop(0, _PPW, position, 0)
    pltpu.make_async_copy(
        rows_v.at[1], out_hbm.at[:, pbase + _PPW - 1, :], ssem
    ).wait()


def kernel(input_ids, table):
    bsz, seq = input_ids.shape
    ids_t = jnp.transpose(input_ids.astype(jnp.int32))
    pe = jnp.asarray(_PE)
    return _emb_kernel(ids_t, table, pe)


# speedup vs baseline: 2.2165x; 1.0163x over previous
import functools
import math

import jax
import jax.numpy as jnp
import numpy as np
from jax import lax
from jax.experimental import pallas as pl
from jax.experimental.pallas import tpu as pltpu
from jax.experimental.pallas import tpu_sc as plsc

_VOCAB = 50268
_DIM = 384
_MAXLEN = 1024
_PAD = 50257
_SCALE = math.sqrt(_DIM)
_BATCH = 64

_NW = 32
_PPW = _MAXLEN // _NW
_LANES = 16
_NSL = _DIM // _LANES


def _build_pe() -> np.ndarray:
    position = np.arange(_MAXLEN, dtype=np.float64)[:, None]
    div_term = np.exp(
        np.arange(0, _DIM, 2, dtype=np.float64) * (-math.log(10000.0) / _DIM)
    )
    pe = np.zeros((_MAXLEN, _DIM), dtype=np.float32)
    pe[:, 0::2] = np.sin(position * div_term)
    pe[:, 1::2] = np.cos(position * div_term)
    return pe


_PE_DIV = _build_pe() / np.float32(_SCALE)

_mesh = plsc.VectorSubcoreMesh(core_axis_name="c", subcore_axis_name="s")


@functools.partial(
    pl.kernel,
    out_type=jax.ShapeDtypeStruct((_BATCH, _MAXLEN, _DIM), jnp.float32),
    mesh=_mesh,
    scratch_types=[
        pltpu.VMEM((_PPW, _DIM), jnp.float32),
        pltpu.VMEM((_PPW, _BATCH), jnp.int32),
        pltpu.VMEM((_PPW * _BATCH + _LANES,), jnp.float32),
        pltpu.VMEM((3, _BATCH, _DIM), jnp.float32),
        pltpu.SemaphoreType.DMA,
        pltpu.SemaphoreType.DMA,
        pltpu.SemaphoreType.DMA,
    ],
)
def _emb_kernel(ids_hbm, table_hbm, pe_hbm, out_hbm,
                pe_v, idx_v, msk_v, rows_v, gsem, ssem0, ssem1):
    wid = lax.axis_index("s") * 2 + lax.axis_index("c")
    pbase = wid * _PPW
    pltpu.sync_copy(ids_hbm.at[pl.ds(pbase, _PPW)], idx_v)
    pltpu.async_copy(table_hbm.at[idx_v.at[0]], rows_v.at[0], gsem)
    pltpu.sync_copy(pe_hbm.at[pl.ds(pbase, _PPW)], pe_v)

    msk_v[pl.ds(_PPW * _BATCH, _LANES)] = jnp.zeros((_LANES,), jnp.float32)

    def mask_row(r, c):
        for c4 in range(_BATCH // _LANES):
            ids16 = idx_v[r, pl.ds(c4 * _LANES, _LANES)]
            msk_v[pl.ds(r * _BATCH + c4 * _LANES, _LANES)] = jnp.where(
                ids16 != _PAD, jnp.float32(_SCALE), jnp.float32(0.0)
            )
        return c

    lax.fori_loop(0, _PPW, mask_row, 0)

    def position(t, carry):
        p = lax.rem(t, 3)
        pn = lax.rem(t + 1, 3)
        pltpu.make_async_copy(
            table_hbm.at[idx_v.at[t]], rows_v.at[p], gsem
        ).wait()

        @pl.when(t >= 2)
        def _():
            @pl.when(lax.rem(t, 2) == 0)
            def _():
                pltpu.make_async_copy(
                    rows_v.at[pn], out_hbm.at[:, pbase + t - 2, :], ssem0
                ).wait()

            @pl.when(lax.rem(t, 2) == 1)
            def _():
                pltpu.make_async_copy(
                    rows_v.at[pn], out_hbm.at[:, pbase + t - 2, :], ssem1
                ).wait()

        @pl.when(t + 1 < _PPW)
        def _():
            pltpu.async_copy(table_hbm.at[idx_v.at[t + 1]], rows_v.at[pn], gsem)

        pe_regs = [pe_v[t, pl.ds(j * _LANES, _LANES)] for j in range(_NSL)]

        def token(b, c):
            m = jnp.full(
                (_LANES,), msk_v[pl.ds(t * _BATCH + b, _LANES)][0], jnp.float32
            )
            for j in range(_NSL):
                sl = pl.ds(j * _LANES, _LANES)
                rows_v[p, b, sl] = (rows_v[p, b, sl] + pe_regs[j]) * m
            return c

        lax.fori_loop(0, _BATCH, token, 0)

        @pl.when(lax.rem(t, 2) == 0)
        def _():
            pltpu.async_copy(rows_v.at[p], out_hbm.at[:, pbase + t, :], ssem0)

        @pl.when(lax.rem(t, 2) == 1)
        def _():
            pltpu.async_copy(rows_v.at[p], out_hbm.at[:, pbase + t, :], ssem1)

        return carry

    lax.fori_loop(0, _PPW, position, 0)
    pltpu.make_async_copy(
        rows_v.at[0], out_hbm.at[:, pbase + _PPW - 2, :], ssem0
    ).wait()
    pltpu.make_async_copy(
        rows_v.at[0], out_hbm.at[:, pbase + _PPW - 1, :], ssem1
    ).wait()


def kernel(input_ids, table):
    bsz, seq = input_ids.shape
    ids_t = jnp.transpose(input_ids.astype(jnp.int32))
    pe = jnp.asarray(_PE_DIV)
    return _emb_kernel(ids_t, table, pe)
